# baseline (speedup 1.0000x reference)
import jax
import jax.numpy as jnp
from jax import lax
from jax.experimental import pallas as pl
from jax.experimental.pallas import tpu as pltpu, tpu_sc as plsc

_WS = 16
_N = _WS * _WS
_NH = 32
_TBL = (2 * _WS - 1) ** 2
_NC, _NS, _L = 2, 16, 16
_NW = _NC * _NS
_RPW = _N // _NW
_BPW = _RPW * _N


_NG = 4
_HPG = _NH // _NG
_WPG = _HPG * _TBL


def _sc_body(table_hbm, idx_hbm, out_hbm, table_v, idx_v, out_v, tsems, osem):
    wid = lax.axis_index("s") * _NC + lax.axis_index("c")
    row0 = wid * _RPW
    tcopies = [
        pltpu.async_copy(
            table_hbm.at[pl.ds(g * _WPG, _WPG)],
            table_v.at[pl.ds(g * _WPG, _WPG)],
            tsems[g],
        )
        for g in range(_NG)
    ]
    pltpu.sync_copy(idx_hbm.at[pl.ds(row0, _RPW), :], idx_v)

    ocopies = []
    for g in range(_NG):
        tcopies[g].wait()

        @plsc.parallel_loop(0, _BPW // _L, unroll=2)
        def _gather_loop(v):
            r = v // (_N // _L)
            c = lax.rem(v, _N // _L)
            pos = idx_v[r, pl.ds(c * _L, _L)]
            for h in range(g * _HPG, (g + 1) * _HPG):
                out_v[h, r, pl.ds(c * _L, _L)] = plsc.load_gather(
                    table_v, [pos + (h * _TBL)]
                )

        ocopies.append(
            pltpu.async_copy(
                out_v.at[pl.ds(g * _HPG, _HPG)],
                out_hbm.at[pl.ds(g * _HPG, _HPG), pl.ds(row0, _RPW), :],
                osem,
            )
        )
    for c in ocopies:
        c.wait()


def kernel(x, relative_position_bias_table, relative_position_index):
    table_flat = relative_position_bias_table.T.reshape(-1)
    mesh = plsc.VectorSubcoreMesh(core_axis_name="c", subcore_axis_name="s")
    out = pl.kernel(
        _sc_body,
        mesh=mesh,
        out_type=jax.ShapeDtypeStruct((_NH, _N, _N), jnp.float32),
        compiler_params=pltpu.CompilerParams(
            needs_layout_passes=False,
            skip_device_barrier=True,
            disable_bounds_checks=True,
            disable_semaphore_checks=True,
        ),
        scratch_types=[
            pltpu.VMEM((_TBL * _NH,), jnp.float32),
            pltpu.VMEM((_RPW, _N), jnp.int32),
            pltpu.VMEM((_NH, _RPW, _N), jnp.float32),
            [pltpu.SemaphoreType.DMA] * _NG,
            pltpu.SemaphoreType.DMA,
        ],
    )(table_flat, relative_position_index)
    return out


# speedup vs baseline: 1.0028x; 1.0028x over previous
import jax
import jax.numpy as jnp
from jax import lax
from jax.experimental import pallas as pl
from jax.experimental.pallas import tpu as pltpu, tpu_sc as plsc

_WS = 16
_N = _WS * _WS
_NH = 32
_TBL = (2 * _WS - 1) ** 2
_NC, _NS, _L = 2, 16, 16
_NW = _NC * _NS
_RPW = _N // _NW
_BPW = _RPW * _N


_NG = 4
_HPG = _NH // _NG
_WPG = _HPG * _TBL


def _sc_body(table_hbm, idx_hbm, out_hbm, table_v, idx_v, out_v, tsems, osem):
    wid = lax.axis_index("s") * _NC + lax.axis_index("c")
    row0 = wid * _RPW
    tcopies = [
        pltpu.async_copy(
            table_hbm.at[pl.ds(g * _WPG, _WPG)],
            table_v.at[pl.ds(g * _WPG, _WPG)],
            tsems[g],
        )
        for g in range(_NG)
    ]
    pltpu.sync_copy(idx_hbm.at[pl.ds(row0, _RPW), :], idx_v)

    ocopies = []
    for g in range(_NG):
        tcopies[g].wait()

        @plsc.parallel_loop(0, _BPW // _L, unroll=2)
        def _gather_loop(v):
            r = v // (_N // _L)
            c = lax.rem(v, _N // _L)
            pos = idx_v[r, pl.ds(c * _L, _L)]
            for h in range(g * _HPG, (g + 1) * _HPG):
                out_v[h, r, pl.ds(c * _L, _L)] = plsc.load_gather(
                    table_v, [pos + (h * _TBL)]
                )

        ocopies.append(
            pltpu.async_copy(
                out_v.at[pl.ds(g * _HPG, _HPG)],
                out_hbm.at[pl.ds(g * _HPG, _HPG), pl.ds(row0, _RPW), :],
                osem,
            )
        )
    for c in ocopies:
        c.wait()


def kernel(x, relative_position_bias_table, relative_position_index):
    table_flat = relative_position_bias_table.T.reshape(-1)
    mesh = plsc.VectorSubcoreMesh(core_axis_name="c", subcore_axis_name="s")
    out = pl.kernel(
        _sc_body,
        mesh=mesh,
        out_type=jax.ShapeDtypeStruct((_NH, _N, _N), jnp.float32),
        compiler_params=pltpu.CompilerParams(needs_layout_passes=False),
        scratch_types=[
            pltpu.VMEM((_TBL * _NH,), jnp.float32),
            pltpu.VMEM((_RPW, _N), jnp.int32),
            pltpu.VMEM((_NH, _RPW, _N), jnp.float32),
            [pltpu.SemaphoreType.DMA] * _NG,
            pltpu.SemaphoreType.DMA,
        ],
    )(table_flat, relative_position_index)
    return out
